# Initial kernel scaffold; baseline (speedup 1.0000x reference)
import jax
import jax.numpy as jnp
from jax.experimental import pallas as pl


def kernel(confidences, wealth):
    raise NotImplementedError("write your pallas kernel here")


# speedup vs baseline: 6.1658x; 6.1658x over previous
import functools

import jax
import jax.numpy as jnp
from jax import lax
from jax.experimental import pallas as pl

TOPK = 8


def _fold_sum(vals):
    vals = list(vals)
    n = len(vals)
    p = 1
    while p < n:
        p *= 2
    while p > 1:
        p //= 2
        nxt = []
        for i in range(p):
            if i + p < len(vals):
                nxt.append(vals[i] + vals[i + p])
            elif i < len(vals):
                nxt.append(vals[i])
        vals = nxt
    return vals[0]


def _auction_body(conf_ref, w_ref, sel_ref, rw_ref, pay_ref):
    conf = conf_ref[...]
    bt, e = conf.shape
    w = w_ref[0:1, :]
    bids = conf * w

    iota = lax.broadcasted_iota(jnp.int32, (bt, e), 1)
    cur = bids
    vals = []
    idxs = []
    for _ in range(TOPK):
        m = jnp.max(cur, axis=-1, keepdims=True)
        eq = cur == m
        idx = jnp.min(jnp.where(eq, iota, e), axis=-1, keepdims=True)
        vals.append(m)
        idxs.append(idx)
        cur = jnp.where(iota == idx, -jnp.inf, cur)

    sel_ref[...] = jnp.concatenate(idxs, axis=-1)

    m0 = vals[0]
    z = jnp.sum(jnp.exp(bids - m0), axis=-1, keepdims=True)
    s = [jnp.exp(v - m0) / z for v in vals]
    denom = _fold_sum(s) + 1e-8
    rw_ref[...] = jnp.concatenate([sj / denom for sj in s], axis=-1)

    s8 = vals[0]
    for v in vals[1:]:
        s8 = s8 + v
    pays = []
    for j in range(TOPK):
        others = vals[:j] + vals[j + 1:]
        s7 = _fold_sum(others)
        pays.append(jnp.maximum(s7 - (s8 - vals[j]), 0.0))
    pay_ref[...] = jnp.concatenate(pays, axis=-1)


@functools.partial(jax.jit, static_argnames=("interpret",))
def _run(confidences, wealth, interpret=False):
    b, s, e = confidences.shape
    t = b * s
    conf2 = confidences.reshape(t, e)
    w2 = jnp.broadcast_to(wealth[None, :], (8, e))
    bt = 2048
    grid = (t // bt,)
    sel, rw, pay = pl.pallas_call(
        _auction_body,
        grid=grid,
        in_specs=[
            pl.BlockSpec((bt, e), lambda i: (i, 0)),
            pl.BlockSpec((8, e), lambda i: (0, 0)),
        ],
        out_specs=[
            pl.BlockSpec((bt, TOPK), lambda i: (i, 0)),
            pl.BlockSpec((bt, TOPK), lambda i: (i, 0)),
            pl.BlockSpec((bt, TOPK), lambda i: (i, 0)),
        ],
        out_shape=[
            jax.ShapeDtypeStruct((t, TOPK), jnp.int32),
            jax.ShapeDtypeStruct((t, TOPK), jnp.float32),
            jax.ShapeDtypeStruct((t, TOPK), jnp.float32),
        ],
        interpret=interpret,
    )(conf2, w2)
    return (sel.reshape(b, s, TOPK), rw.reshape(b, s, TOPK),
            pay.reshape(b, s, TOPK))


def kernel(confidences, wealth):
    return _run(confidences, wealth)
